# baseline (speedup 1.0000x reference)
import functools

import jax
import jax.numpy as jnp
from jax import lax
from jax.experimental import pallas as pl
from jax.experimental.pallas import tpu as pltpu
from jax.experimental.pallas import tpu_sc as plsc

B, T, D_IN = 16, 2048, 512
K, D_C = 1024, 256
N = B * T

R_A = 1024
NKC = 4
CK = K // NKC
SC_WORKERS = 32
SC_CHUNK = 128
_PREC = lax.Precision.DEFAULT


def _stage_a_body(x_ref, in_v_ref, in_g_ref, in_b_ref, cb_ref, cs_ref, cbias_ref,
                  z_e_ref, idx_ref, w_in_s, cbs_s, cbsq_s):
    i = pl.program_id(0)

    @pl.when(i == 0)
    def _prep():
        v = in_v_ref[...]
        nrm = jnp.sqrt(jnp.sum(v * v, axis=0, keepdims=True))
        w_in_s[...] = in_g_ref[...] * v / jnp.maximum(nrm, 1e-12)
        cb = cb_ref[...]
        cn = jnp.sqrt(jnp.sum(cb * cb, axis=1, keepdims=True))
        cbs = cb / jnp.maximum(cn, 1e-12) * cs_ref[...] + cbias_ref[...]
        cbs_s[...] = cbs
        ones = jnp.ones((1, D_C), jnp.float32)
        cbsq_s[...] = lax.dot_general(ones, cbs * cbs, (((1,), (1,)), ((), ())),
                                      precision=lax.Precision.HIGHEST,
                                      preferred_element_type=jnp.float32)

    x_t = x_ref[...]
    z_e = lax.dot_general(x_t, w_in_s[...], (((1,), (0,)), ((), ())),
                          precision=_PREC, preferred_element_type=jnp.float32)
    z_e = z_e + in_b_ref[...]
    z_e_ref[...] = z_e
    nrm = jnp.sqrt(jnp.sum(z_e * z_e, axis=1, keepdims=True))
    enc = z_e / jnp.maximum(nrm, 1e-12)
    rs = jnp.sum(enc * enc, axis=1, keepdims=True)
    mm = lax.dot_general(enc, cbs_s[...], (((1,), (1,)), ((), ())),
                         precision=_PREC, preferred_element_type=jnp.float32)
    d = rs - 2.0 * mm + cbsq_s[...]
    m = jnp.min(d, axis=1, keepdims=True)
    maskf = jnp.where(d == m, jnp.float32(1.0), jnp.float32(0.0))
    iota_col = lax.broadcasted_iota(jnp.int32, (K, 1), 0).astype(jnp.float32)
    idxf = lax.dot_general(maskf, iota_col, (((1,), (0,)), ((), ())),
                           precision=lax.Precision.HIGHEST,
                           preferred_element_type=jnp.float32)
    idx_ref[...] = jnp.minimum(idxf, jnp.float32(K - 1)).astype(jnp.int32)


def _stage_a(x2, in_v, in_g2, in_b2, codebook, cs2, cbias2):
    grid = (N // R_A,)
    return pl.pallas_call(
        _stage_a_body,
        grid=grid,
        in_specs=[
            pl.BlockSpec((R_A, D_IN), lambda i: (i, 0)),
            pl.BlockSpec((D_IN, D_C), lambda i: (0, 0)),
            pl.BlockSpec((1, D_C), lambda i: (0, 0)),
            pl.BlockSpec((1, D_C), lambda i: (0, 0)),
            pl.BlockSpec((K, D_C), lambda i: (0, 0)),
            pl.BlockSpec((1, D_C), lambda i: (0, 0)),
            pl.BlockSpec((1, D_C), lambda i: (0, 0)),
        ],
        out_specs=[
            pl.BlockSpec((R_A, D_C), lambda i: (i, 0)),
            pl.BlockSpec((R_A, 1), lambda i: (i, 0)),
        ],
        out_shape=[
            jax.ShapeDtypeStruct((N, D_C), jnp.float32),
            jax.ShapeDtypeStruct((N, 1), jnp.int32),
        ],
        scratch_shapes=[
            pltpu.VMEM((D_IN, D_C), jnp.float32),
            pltpu.VMEM((K, D_C), jnp.float32),
            pltpu.VMEM((1, K), jnp.float32),
        ],
    )(x2, in_v, in_g2, in_b2, codebook, cs2, cbias2)


def _sc_gather(codebook, idx_flat):
    b_per_w = N // SC_WORKERS
    n_chunks = b_per_w // SC_CHUNK
    mesh = plsc.VectorSubcoreMesh(core_axis_name="c", subcore_axis_name="s")

    @functools.partial(
        pl.kernel,
        out_type=jax.ShapeDtypeStruct((N, D_C), jnp.float32),
        mesh=mesh,
        scratch_types=[
            pltpu.VMEM((b_per_w,), jnp.int32),
            pltpu.VMEM((SC_CHUNK, D_C), jnp.float32),
            pltpu.VMEM((SC_CHUNK, D_C), jnp.float32),
            pltpu.SemaphoreType.DMA,
            pltpu.SemaphoreType.DMA,
            pltpu.SemaphoreType.DMA,
            pltpu.SemaphoreType.DMA,
        ],
    )
    def gather_kernel(table_hbm, idx_hbm, out_hbm, idx_v, rows0, rows1,
                      gsem0, gsem1, osem0, osem1):
        wid = lax.axis_index("s") * 2 + lax.axis_index("c")
        base_w = wid * b_per_w
        pltpu.sync_copy(idx_hbm.at[pl.ds(base_w, b_per_w)], idx_v)
        rows = (rows0, rows1)
        gsem = (gsem0, gsem1)
        osem = (osem0, osem1)

        def start_gather(c):
            b = c % 2
            return pltpu.async_copy(
                table_hbm.at[idx_v.at[pl.ds(c * SC_CHUNK, SC_CHUNK)]],
                rows[b], gsem[b])

        g = [start_gather(0), None]
        o = [None, None]
        for c in range(n_chunks):
            b = c % 2
            nb = (c + 1) % 2
            if c + 1 < n_chunks:
                if o[nb] is not None:
                    o[nb].wait()
                g[nb] = start_gather(c + 1)
            g[b].wait()
            o[b] = pltpu.async_copy(
                rows[b], out_hbm.at[pl.ds(base_w + c * SC_CHUNK, SC_CHUNK)],
                osem[b])
        o[(n_chunks - 1) % 2].wait()
        if n_chunks > 1:
            o[n_chunks % 2].wait()

    return gather_kernel(codebook, idx_flat)


def _stage_b_body(z_e_ref, z_q_ref, out_v_ref, out_g_ref, out_b_ref,
                  z_q_out_ref, loss_ref, w_out_s):
    i = pl.program_id(0)

    @pl.when(i == 0)
    def _prep():
        v = out_v_ref[...]
        nrm = jnp.sqrt(jnp.sum(v * v, axis=0, keepdims=True))
        w_out_s[...] = out_g_ref[...] * v / jnp.maximum(nrm, 1e-12)

    z_q = z_q_ref[...]
    z_e = z_e_ref[...]
    out = lax.dot_general(z_q, w_out_s[...], (((1,), (0,)), ((), ())),
                          precision=_PREC, preferred_element_type=jnp.float32)
    z_q_out_ref[...] = out + out_b_ref[...]
    diff = z_e - z_q
    loss_ref[...] = jnp.reshape(jnp.sum(diff * diff) * (1.0 / (T * D_C)), (1, 1, 1))


def _stage_b(z_e2, z_q2, out_v, out_g2, out_b2):
    grid = (B,)
    return pl.pallas_call(
        _stage_b_body,
        grid=grid,
        in_specs=[
            pl.BlockSpec((T, D_C), lambda i: (i, 0)),
            pl.BlockSpec((T, D_C), lambda i: (i, 0)),
            pl.BlockSpec((D_C, D_IN), lambda i: (0, 0)),
            pl.BlockSpec((1, D_IN), lambda i: (0, 0)),
            pl.BlockSpec((1, D_IN), lambda i: (0, 0)),
        ],
        out_specs=[
            pl.BlockSpec((T, D_IN), lambda i: (i, 0)),
            pl.BlockSpec((1, 1, 1), lambda i: (i, 0, 0)),
        ],
        out_shape=[
            jax.ShapeDtypeStruct((N, D_IN), jnp.float32),
            jax.ShapeDtypeStruct((B, 1, 1), jnp.float32),
        ],
        scratch_shapes=[
            pltpu.VMEM((D_C, D_IN), jnp.float32),
        ],
    )(z_e2, z_q2, out_v, out_g2, out_b2)


def kernel(x, in_v, in_g, in_b, out_v, out_g, out_b, codebook, code_scale, code_bias):
    x2 = x.reshape(N, D_IN)
    z_e2, idx2 = _stage_a(x2, in_v, in_g.reshape(1, D_C), in_b.reshape(1, D_C),
                          codebook, code_scale.reshape(1, D_C),
                          code_bias.reshape(1, D_C))
    idx_flat = idx2.reshape(N)
    z_q2 = _sc_gather(codebook, idx_flat)
    z_q_out2, loss3 = _stage_b(z_e2, z_q2, out_v, out_g.reshape(1, D_IN),
                               out_b.reshape(1, D_IN))
    loss = loss3.reshape(B)
    return (z_q_out2.reshape(B, T, D_IN), loss, loss,
            idx_flat.reshape(B, T), z_e2.reshape(B, T, D_C))


# speedup vs baseline: 1.6427x; 1.6427x over previous
import functools

import jax
import jax.numpy as jnp
from jax import lax
from jax.experimental import pallas as pl
from jax.experimental.pallas import tpu as pltpu
from jax.experimental.pallas import tpu_sc as plsc

B, T, D_IN = 16, 2048, 512
K, D_C = 1024, 256
N = B * T

R_A = 1024
NKC = 4
CK = K // NKC
SC_WORKERS = 32
SC_CHUNK = 128
_PREC = lax.Precision.DEFAULT


def _stage_a_body(x_ref, in_v_ref, in_g_ref, in_b_ref, cb_ref, cs_ref, cbias_ref,
                  z_e_ref, idx_ref, w_in_s, cbs_s, cbsq_s, d_s):
    i = pl.program_id(0)

    @pl.when(i == 0)
    def _prep():
        v = in_v_ref[...]
        nrm = jnp.sqrt(jnp.sum(v * v, axis=0, keepdims=True))
        w_in_s[...] = in_g_ref[...] * v / jnp.maximum(nrm, 1e-12)
        cb = cb_ref[...]
        cn = jnp.sqrt(jnp.sum(cb * cb, axis=1, keepdims=True))
        cbs = cb / jnp.maximum(cn, 1e-12) * cs_ref[...] + cbias_ref[...]
        cbs_s[...] = cbs
        ones = jnp.ones((1, D_C), jnp.float32)
        cbsq_s[...] = lax.dot_general(ones, cbs * cbs, (((1,), (1,)), ((), ())),
                                      precision=lax.Precision.HIGHEST,
                                      preferred_element_type=jnp.float32)

    x_t = x_ref[...]
    z_e = lax.dot_general(x_t, w_in_s[...], (((1,), (0,)), ((), ())),
                          precision=_PREC, preferred_element_type=jnp.float32)
    z_e = z_e + in_b_ref[...]
    z_e_ref[...] = z_e
    nrm = jnp.sqrt(jnp.sum(z_e * z_e, axis=1, keepdims=True))
    enc = z_e / jnp.maximum(nrm, 1e-12)
    rs = jnp.sum(enc * enc, axis=1, keepdims=True)
    for j in range(NKC):
        sl = pl.ds(j * CK, CK)
        mm_j = lax.dot_general(enc, cbs_s[sl, :], (((1,), (1,)), ((), ())),
                               precision=_PREC, preferred_element_type=jnp.float32)
        d_s[:, sl] = rs - 2.0 * mm_j + cbsq_s[:, sl]
    d = d_s[...]
    m = jnp.min(d, axis=1, keepdims=True)
    kid = lax.broadcasted_iota(jnp.int32, (R_A, K), 1)
    idx_ref[...] = jnp.min(jnp.where(d == m, kid, jnp.int32(K)),
                           axis=1, keepdims=True)


def _stage_a(x2, in_v, in_g2, in_b2, codebook, cs2, cbias2):
    grid = (N // R_A,)
    return pl.pallas_call(
        _stage_a_body,
        grid=grid,
        in_specs=[
            pl.BlockSpec((R_A, D_IN), lambda i: (i, 0)),
            pl.BlockSpec((D_IN, D_C), lambda i: (0, 0)),
            pl.BlockSpec((1, D_C), lambda i: (0, 0)),
            pl.BlockSpec((1, D_C), lambda i: (0, 0)),
            pl.BlockSpec((K, D_C), lambda i: (0, 0)),
            pl.BlockSpec((1, D_C), lambda i: (0, 0)),
            pl.BlockSpec((1, D_C), lambda i: (0, 0)),
        ],
        out_specs=[
            pl.BlockSpec((R_A, D_C), lambda i: (i, 0)),
            pl.BlockSpec((R_A, 1), lambda i: (i, 0)),
        ],
        out_shape=[
            jax.ShapeDtypeStruct((N, D_C), jnp.float32),
            jax.ShapeDtypeStruct((N, 1), jnp.int32),
        ],
        scratch_shapes=[
            pltpu.VMEM((D_IN, D_C), jnp.float32),
            pltpu.VMEM((K, D_C), jnp.float32),
            pltpu.VMEM((1, K), jnp.float32),
            pltpu.VMEM((R_A, K), jnp.float32),
        ],
    )(x2, in_v, in_g2, in_b2, codebook, cs2, cbias2)


def _sc_gather(codebook, idx_flat):
    b_per_w = N // SC_WORKERS
    n_chunks = b_per_w // SC_CHUNK
    mesh = plsc.VectorSubcoreMesh(core_axis_name="c", subcore_axis_name="s")

    @functools.partial(
        pl.kernel,
        out_type=jax.ShapeDtypeStruct((N, D_C), jnp.float32),
        mesh=mesh,
        scratch_types=[
            pltpu.VMEM((b_per_w,), jnp.int32),
            pltpu.VMEM((SC_CHUNK, D_C), jnp.float32),
            pltpu.VMEM((SC_CHUNK, D_C), jnp.float32),
            pltpu.SemaphoreType.DMA,
            pltpu.SemaphoreType.DMA,
            pltpu.SemaphoreType.DMA,
            pltpu.SemaphoreType.DMA,
        ],
    )
    def gather_kernel(table_hbm, idx_hbm, out_hbm, idx_v, rows0, rows1,
                      gsem0, gsem1, osem0, osem1):
        wid = lax.axis_index("s") * 2 + lax.axis_index("c")
        base_w = wid * b_per_w
        pltpu.sync_copy(idx_hbm.at[pl.ds(base_w, b_per_w)], idx_v)
        rows = (rows0, rows1)
        gsem = (gsem0, gsem1)
        osem = (osem0, osem1)

        def start_gather(c):
            b = c % 2
            return pltpu.async_copy(
                table_hbm.at[idx_v.at[pl.ds(c * SC_CHUNK, SC_CHUNK)]],
                rows[b], gsem[b])

        g = [start_gather(0), None]
        o = [None, None]
        for c in range(n_chunks):
            b = c % 2
            nb = (c + 1) % 2
            if c + 1 < n_chunks:
                if o[nb] is not None:
                    o[nb].wait()
                g[nb] = start_gather(c + 1)
            g[b].wait()
            o[b] = pltpu.async_copy(
                rows[b], out_hbm.at[pl.ds(base_w + c * SC_CHUNK, SC_CHUNK)],
                osem[b])
        o[(n_chunks - 1) % 2].wait()
        if n_chunks > 1:
            o[n_chunks % 2].wait()

    return gather_kernel(codebook, idx_flat)


def _stage_b_body(z_e_ref, z_q_ref, out_v_ref, out_g_ref, out_b_ref,
                  z_q_out_ref, loss_ref, w_out_s):
    i = pl.program_id(0)

    @pl.when(i == 0)
    def _prep():
        v = out_v_ref[...]
        nrm = jnp.sqrt(jnp.sum(v * v, axis=0, keepdims=True))
        w_out_s[...] = out_g_ref[...] * v / jnp.maximum(nrm, 1e-12)

    z_q = z_q_ref[...]
    z_e = z_e_ref[...]
    out = lax.dot_general(z_q, w_out_s[...], (((1,), (0,)), ((), ())),
                          precision=_PREC, preferred_element_type=jnp.float32)
    z_q_out_ref[...] = out + out_b_ref[...]
    diff = z_e - z_q
    loss_ref[...] = jnp.reshape(jnp.sum(diff * diff) * (1.0 / (T * D_C)), (1, 1, 1))


def _stage_b(z_e2, z_q2, out_v, out_g2, out_b2):
    grid = (B,)
    return pl.pallas_call(
        _stage_b_body,
        grid=grid,
        in_specs=[
            pl.BlockSpec((T, D_C), lambda i: (i, 0)),
            pl.BlockSpec((T, D_C), lambda i: (i, 0)),
            pl.BlockSpec((D_C, D_IN), lambda i: (0, 0)),
            pl.BlockSpec((1, D_IN), lambda i: (0, 0)),
            pl.BlockSpec((1, D_IN), lambda i: (0, 0)),
        ],
        out_specs=[
            pl.BlockSpec((T, D_IN), lambda i: (i, 0)),
            pl.BlockSpec((1, 1, 1), lambda i: (i, 0, 0)),
        ],
        out_shape=[
            jax.ShapeDtypeStruct((N, D_IN), jnp.float32),
            jax.ShapeDtypeStruct((B, 1, 1), jnp.float32),
        ],
        scratch_shapes=[
            pltpu.VMEM((D_C, D_IN), jnp.float32),
        ],
    )(z_e2, z_q2, out_v, out_g2, out_b2)


def kernel(x, in_v, in_g, in_b, out_v, out_g, out_b, codebook, code_scale, code_bias):
    x2 = x.reshape(N, D_IN)
    z_e2, idx2 = _stage_a(x2, in_v, in_g.reshape(1, D_C), in_b.reshape(1, D_C),
                          codebook, code_scale.reshape(1, D_C),
                          code_bias.reshape(1, D_C))
    idx_flat = idx2.reshape(N)
    z_q2 = _sc_gather(codebook, idx_flat)
    z_q_out2, loss3 = _stage_b(z_e2, z_q2, out_v, out_g.reshape(1, D_IN),
                               out_b.reshape(1, D_IN))
    loss = loss3.reshape(B)
    return (z_q_out2.reshape(B, T, D_IN), loss, loss,
            idx_flat.reshape(B, T), z_e2.reshape(B, T, D_C))
